# baseline (speedup 1.0000x reference)
import functools

import jax
import jax.numpy as jnp
from jax import lax
from jax.experimental import pallas as pl
from jax.experimental.pallas import tpu as pltpu
from jax.experimental.pallas import tpu_sc as plsc


def _sc_gather(table, idx_pad, b, s, sp):
    d = table.shape[1]
    info = plsc.get_sparse_core_info()
    nc = info.num_cores
    nw = nc * info.num_subcores
    bpw = b // nw
    assert bpw * nw == b

    mesh = plsc.VectorSubcoreMesh(core_axis_name="c", subcore_axis_name="s")

    @functools.partial(
        pl.kernel,
        mesh=mesh,
        out_type=jax.ShapeDtypeStruct((b, s, d), jnp.float32),
        scratch_types=[
            pltpu.VMEM((bpw * sp,), jnp.int32),
            pltpu.VMEM((2, sp, d), jnp.float32),
            pltpu.SemaphoreType.DMA,
            pltpu.SemaphoreType.DMA,
        ],
    )
    def k(table_hbm, idx_hbm, out_hbm, idx_v, buf, gsem, ssem):
        wid = lax.axis_index("s") * nc + lax.axis_index("c")
        b0 = wid * bpw
        pltpu.sync_copy(idx_hbm.at[pl.ds(b0 * sp, bpw * sp)], idx_v)

        def start_gather(c, slot):
            pltpu.async_copy(
                table_hbm.at[idx_v.at[pl.ds(c * sp, sp)]], buf.at[slot], gsem)

        def wait_gather(c, slot):
            pltpu.make_async_copy(
                table_hbm.at[idx_v.at[pl.ds(c * sp, sp)]], buf.at[slot],
                gsem).wait()

        def start_scatter(c, slot):
            pltpu.async_copy(
                buf.at[slot, pl.ds(0, s)], out_hbm.at[b0 + c], ssem)

        def wait_scatter():
            pltpu.make_async_copy(
                buf.at[0, pl.ds(0, s)], out_hbm.at[b0], ssem).wait()

        start_gather(0, 0)

        def step(c):
            slot = lax.rem(c, 2)
            wait_gather(c, slot)
            @pl.when(c >= 1)
            def _():
                wait_scatter()

            @pl.when(c + 1 < bpw)
            def _():
                start_gather(c + 1, 1 - slot)

            start_scatter(c, slot)

        pl.loop(0, bpw)(step)
        wait_scatter()

    return k(table, idx_pad)


def _tc_ln(tok, pos_table, seg_table, gamma, beta, max_sent):
    b, s, d = tok.shape
    bb = 8
    assert b % bb == 0

    def body(tok_ref, pos_ref, seg_ref, g_ref, b_ref, o_ref):
        h = tok_ref[...] + pos_ref[...][None, :, :]
        row = lax.broadcasted_iota(jnp.int32, (1, s, 1), 1)
        segv = jnp.where(row < max_sent + 1, seg_ref[0][None, None, :],
                         seg_ref[1][None, None, :])
        h = h + segv
        mean = jnp.mean(h, axis=-1, keepdims=True)
        c = h - mean
        var = jnp.mean(c * c, axis=-1, keepdims=True)
        o_ref[...] = (c * lax.rsqrt(var + 1e-5)) * g_ref[...] + b_ref[...]

    return pl.pallas_call(
        body,
        grid=(b // bb,),
        in_specs=[
            pl.BlockSpec((bb, s, d), lambda i: (i, 0, 0)),
            pl.BlockSpec((s, d), lambda i: (0, 0)),
            pl.BlockSpec((2, d), lambda i: (0, 0)),
            pl.BlockSpec((d,), lambda i: (0,)),
            pl.BlockSpec((d,), lambda i: (0,)),
        ],
        out_specs=pl.BlockSpec((bb, s, d), lambda i: (i, 0, 0)),
        out_shape=jax.ShapeDtypeStruct((b, s, d), jnp.float32),
    )(tok, pos_table, seg_table, gamma, beta)


def kernel(x, token_table, pos_table, seg_table, gamma, beta):
    b, s = x.shape
    d = token_table.shape[1]
    max_sent = (s - 2) // 2
    sp = (s + 7) // 8 * 8
    idx_pad = jnp.pad(x.astype(jnp.int32), ((0, 0), (0, sp - s))).reshape(-1)
    tok = _sc_gather(token_table, idx_pad, b, s, sp)
    return _tc_ln(tok, pos_table, seg_table, gamma, beta, max_sent)


# speedup vs baseline: 1.0029x; 1.0029x over previous
import functools

import jax
import jax.numpy as jnp
from jax import lax
from jax.experimental import pallas as pl
from jax.experimental.pallas import tpu as pltpu
from jax.experimental.pallas import tpu_sc as plsc


def _sc_gather(table, idx_pad, b, s, sp):
    d = table.shape[1]
    info = plsc.get_sparse_core_info()
    nc = info.num_cores
    nw = nc * info.num_subcores
    bpw = b // nw
    assert bpw * nw == b

    mesh = plsc.VectorSubcoreMesh(core_axis_name="c", subcore_axis_name="s")

    @functools.partial(
        pl.kernel,
        mesh=mesh,
        out_type=jax.ShapeDtypeStruct((b, s, d), jnp.float32),
        scratch_types=[
            pltpu.VMEM((bpw * sp,), jnp.int32),
            pltpu.VMEM((2, sp, d), jnp.float32),
            pltpu.SemaphoreType.DMA,
            pltpu.SemaphoreType.DMA,
        ],
    )
    def k(table_hbm, idx_hbm, out_hbm, idx_v, buf, gsem, ssem):
        wid = lax.axis_index("s") * nc + lax.axis_index("c")
        b0 = wid * bpw
        pltpu.sync_copy(idx_hbm.at[pl.ds(b0 * sp, bpw * sp)], idx_v)

        def start_gather(c, slot):
            pltpu.async_copy(
                table_hbm.at[idx_v.at[pl.ds(c * sp, sp)]], buf.at[slot], gsem)

        def wait_gather(c, slot):
            pltpu.make_async_copy(
                table_hbm.at[idx_v.at[pl.ds(c * sp, sp)]], buf.at[slot],
                gsem).wait()

        def start_scatter(c, slot):
            pltpu.async_copy(
                buf.at[slot, pl.ds(0, s)], out_hbm.at[b0 + c], ssem)

        def wait_scatter():
            pltpu.make_async_copy(
                buf.at[0, pl.ds(0, s)], out_hbm.at[b0], ssem).wait()

        def step(c):
            slot = lax.rem(c, 2)
            start_gather(c, slot)
            wait_gather(c, slot)
            start_scatter(c, slot)
            wait_scatter()

        pl.loop(0, bpw)(step)

    return k(table, idx_pad)


def _tc_ln(tok, pos_table, seg_table, gamma, beta, max_sent):
    b, s, d = tok.shape
    bb = 8
    assert b % bb == 0

    def body(tok_ref, pos_ref, seg_ref, g_ref, b_ref, o_ref):
        h = tok_ref[...] + pos_ref[...][None, :, :]
        row = lax.broadcasted_iota(jnp.int32, (1, s, 1), 1)
        segv = jnp.where(row < max_sent + 1, seg_ref[0][None, None, :],
                         seg_ref[1][None, None, :])
        h = h + segv
        mean = jnp.mean(h, axis=-1, keepdims=True)
        c = h - mean
        var = jnp.mean(c * c, axis=-1, keepdims=True)
        o_ref[...] = (c * lax.rsqrt(var + 1e-5)) * g_ref[...] + b_ref[...]

    return pl.pallas_call(
        body,
        grid=(b // bb,),
        in_specs=[
            pl.BlockSpec((bb, s, d), lambda i: (i, 0, 0)),
            pl.BlockSpec((s, d), lambda i: (0, 0)),
            pl.BlockSpec((2, d), lambda i: (0, 0)),
            pl.BlockSpec((d,), lambda i: (0,)),
            pl.BlockSpec((d,), lambda i: (0,)),
        ],
        out_specs=pl.BlockSpec((bb, s, d), lambda i: (i, 0, 0)),
        out_shape=jax.ShapeDtypeStruct((b, s, d), jnp.float32),
    )(tok, pos_table, seg_table, gamma, beta)


def kernel(x, token_table, pos_table, seg_table, gamma, beta):
    b, s = x.shape
    d = token_table.shape[1]
    max_sent = (s - 2) // 2
    sp = (s + 7) // 8 * 8
    idx_pad = jnp.pad(x.astype(jnp.int32), ((0, 0), (0, sp - s))).reshape(-1)
    tok = _sc_gather(token_table, idx_pad, b, s, sp)
    return _tc_ln(tok, pos_table, seg_table, gamma, beta, max_sent)


# speedup vs baseline: 2.6664x; 2.6588x over previous
import functools

import jax
import jax.numpy as jnp
from jax import lax
from jax.experimental import pallas as pl
from jax.experimental.pallas import tpu as pltpu
from jax.experimental.pallas import tpu_sc as plsc


def _sc_gather(table, idx_flat):
    n = idx_flat.shape[0]
    d = table.shape[1]
    info = plsc.get_sparse_core_info()
    nc = info.num_cores
    nw = nc * info.num_subcores
    per_w = n // nw
    ch = 192
    n_ch = per_w // ch
    assert per_w * nw == n and n_ch * ch == per_w

    mesh = plsc.VectorSubcoreMesh(core_axis_name="c", subcore_axis_name="s")

    @functools.partial(
        pl.kernel,
        mesh=mesh,
        out_type=jax.ShapeDtypeStruct((n, d), jnp.float32),
        scratch_types=[
            pltpu.VMEM((per_w,), jnp.int32),
            pltpu.VMEM((2, ch, d), jnp.float32),
            pltpu.SemaphoreType.DMA,
            pltpu.SemaphoreType.DMA,
        ],
    )
    def k(table_hbm, idx_hbm, out_hbm, idx_v, buf, gsem, ssem):
        wid = lax.axis_index("s") * nc + lax.axis_index("c")
        base = wid * per_w
        pltpu.sync_copy(idx_hbm.at[pl.ds(base, per_w)], idx_v)

        def start_gather(c, slot):
            pltpu.async_copy(
                table_hbm.at[idx_v.at[pl.ds(c * ch, ch)]], buf.at[slot], gsem)

        def wait_gather(c, slot):
            pltpu.make_async_copy(
                table_hbm.at[idx_v.at[pl.ds(c * ch, ch)]], buf.at[slot],
                gsem).wait()

        def start_scatter(c, slot):
            pltpu.async_copy(
                buf.at[slot], out_hbm.at[pl.ds(base + c * ch, ch)], ssem)

        def wait_scatter():
            pltpu.make_async_copy(
                buf.at[0], out_hbm.at[pl.ds(base, ch)], ssem).wait()

        start_gather(0, 0)

        def step(c):
            slot = lax.rem(c, 2)
            wait_gather(c, slot)
            @pl.when(c >= 1)
            def _():
                wait_scatter()

            @pl.when(c + 1 < n_ch)
            def _():
                start_gather(c + 1, 1 - slot)

            start_scatter(c, slot)

        pl.loop(0, n_ch)(step)
        wait_scatter()

    return k(table, idx_flat)


def _tc_ln(tok_t, pos_table, seg_table, gamma, beta, max_sent):
    s, b, d = tok_t.shape
    sb = 6
    assert s % sb == 0

    def body(tok_ref, pos_ref, seg_ref, g_ref, b_ref, o_ref):
        i = pl.program_id(0)
        h = tok_ref[...] + pos_ref[...]
        srow = lax.broadcasted_iota(jnp.int32, (sb, 1, 1), 0) + i * sb
        segv = jnp.where(srow < max_sent + 1, seg_ref[0][None, None, :],
                         seg_ref[1][None, None, :])
        h = h + segv
        mean = jnp.mean(h, axis=-1, keepdims=True)
        c = h - mean
        var = jnp.mean(c * c, axis=-1, keepdims=True)
        o_ref[...] = (c * lax.rsqrt(var + 1e-5)) * g_ref[...] + b_ref[...]

    return pl.pallas_call(
        body,
        grid=(s // sb,),
        in_specs=[
            pl.BlockSpec((sb, b, d), lambda i: (i, 0, 0)),
            pl.BlockSpec((sb, 1, d), lambda i: (i, 0, 0)),
            pl.BlockSpec((2, d), lambda i: (0, 0)),
            pl.BlockSpec((d,), lambda i: (0,)),
            pl.BlockSpec((d,), lambda i: (0,)),
        ],
        out_specs=pl.BlockSpec((sb, b, d), lambda i: (i, 0, 0)),
        out_shape=jax.ShapeDtypeStruct((s, b, d), jnp.float32),
    )(tok_t, pos_table.reshape(s, 1, d), seg_table, gamma, beta)


def kernel(x, token_table, pos_table, seg_table, gamma, beta):
    b, s = x.shape
    d = token_table.shape[1]
    max_sent = (s - 2) // 2
    idx_t = x.T.astype(jnp.int32).reshape(-1)
    tok = _sc_gather(token_table, idx_t)
    out_t = _tc_ln(tok.reshape(s, b, d), pos_table, seg_table, gamma, beta,
                   max_sent)
    return jnp.transpose(out_t, (1, 0, 2))
